# baseline (speedup 1.0000x reference)
import functools

import jax
import jax.numpy as jnp
from jax import lax
from jax.experimental import pallas as pl
from jax.experimental.pallas import tpu as pltpu
from jax.experimental.pallas import tpu_sc as plsc

BATCH = 16384
D = 32
NC = 2
NS = 16
NW = NC * NS
BPW = BATCH // NW
GCH = 128
NG = BPW // GCH

_mesh = plsc.VectorSubcoreMesh(core_axis_name="c", subcore_axis_name="s")


@functools.partial(
    pl.kernel,
    mesh=_mesh,
    compiler_params=pltpu.CompilerParams(needs_layout_passes=False,
                                         use_tc_tiling_on_sc=False),
    out_type=jax.ShapeDtypeStruct((NW, BPW), jnp.float32),
    scratch_types=[
        pltpu.VMEM((NG, GCH), jnp.int32),
        pltpu.VMEM((NG, GCH), jnp.int32),
        pltpu.VMEM((BPW, D), jnp.float32),
        pltpu.VMEM((BPW, D), jnp.float32),
        pltpu.VMEM((NS * BPW,), jnp.float32),
        pltpu.VMEM((BPW,), jnp.float32),
        pltpu.SemaphoreType.DMA,
        pltpu.SemaphoreType.DMA,
    ],
)
def _mf_kernel(uidx_hbm, midx_hbm, utab_hbm, mtab_hbm, out_hbm,
               uidx_v, midx_v, urows_v, mrows_v, pt_v, out_v, sem_u, sem_m):
    wid = lax.axis_index("s") * NC + lax.axis_index("c")

    pltpu.sync_copy(uidx_hbm.at[wid], uidx_v)
    pltpu.sync_copy(midx_hbm.at[wid], midx_v)

    for g in range(NG):
        pltpu.async_copy(utab_hbm.at[uidx_v.at[g]],
                         urows_v.at[pl.ds(g * GCH, GCH)], sem_u)
        pltpu.async_copy(mtab_hbm.at[midx_v.at[g]],
                         mrows_v.at[pl.ds(g * GCH, GCH)], sem_m)
    for g in range(NG):
        pltpu.make_async_copy(utab_hbm.at[uidx_v.at[g]],
                              urows_v.at[pl.ds(g * GCH, GCH)], sem_u).wait()
        pltpu.make_async_copy(mtab_hbm.at[midx_v.at[g]],
                              mrows_v.at[pl.ds(g * GCH, GCH)], sem_m).wait()

    lanes = lax.iota(jnp.int32, 16)

    def row_body(b, carry):
        s = (urows_v[b, pl.ds(0, 16)] * mrows_v[b, pl.ds(0, 16)]
             + urows_v[b, pl.ds(16, 16)] * mrows_v[b, pl.ds(16, 16)])
        plsc.store_scatter(pt_v, [lanes * BPW + b], s)
        return carry

    lax.fori_loop(0, BPW, row_body, 0)

    def red_body(gidx, carry):
        col = gidx * 16
        acc = pt_v[pl.ds(col, 16)]
        for dd in range(1, NS):
            acc = acc + pt_v[pl.ds(dd * BPW + col, 16)]
        out_v[pl.ds(col, 16)] = acc
        return carry

    lax.fori_loop(0, BPW // 16, red_body, 0)

    pltpu.sync_copy(out_v, out_hbm.at[wid])


def kernel(user_idx, movie_idx, user_table, movie_table):
    uidx = user_idx.reshape(NW, NG, GCH).astype(jnp.int32)
    midx = movie_idx.reshape(NW, NG, GCH).astype(jnp.int32)
    out = _mf_kernel(uidx, midx, user_table, movie_table)
    return out.reshape(BATCH)


# speedup vs baseline: 1.0012x; 1.0012x over previous
import functools

import jax
import jax.numpy as jnp
from jax import lax
from jax.experimental import pallas as pl
from jax.experimental.pallas import tpu as pltpu
from jax.experimental.pallas import tpu_sc as plsc

BATCH = 16384
D = 32
NC = 2
NS = 16
NW = NC * NS
BPW = BATCH // NW
GCH = 128
NG = BPW // GCH

_mesh = plsc.VectorSubcoreMesh(core_axis_name="c", subcore_axis_name="s")


@functools.partial(
    pl.kernel,
    mesh=_mesh,
    compiler_params=pltpu.CompilerParams(needs_layout_passes=False,
                                         use_tc_tiling_on_sc=False),
    out_type=jax.ShapeDtypeStruct((BATCH,), jnp.float32),
    scratch_types=[
        pltpu.VMEM((BPW,), jnp.int32),
        pltpu.VMEM((BPW,), jnp.int32),
        pltpu.VMEM((BPW, D), jnp.float32),
        pltpu.VMEM((BPW, D), jnp.float32),
        pltpu.VMEM((NS * BPW,), jnp.float32),
        pltpu.VMEM((BPW,), jnp.float32),
        pltpu.SemaphoreType.DMA,
        pltpu.SemaphoreType.DMA,
    ],
)
def _mf_kernel(uidx_hbm, midx_hbm, utab_hbm, mtab_hbm, out_hbm,
               uidx_v, midx_v, urows_v, mrows_v, pt_v, out_v, sem_u, sem_m):
    wid = lax.axis_index("s") * NC + lax.axis_index("c")
    base = wid * BPW

    pltpu.sync_copy(uidx_hbm.at[pl.ds(base, BPW)], uidx_v)
    pltpu.sync_copy(midx_hbm.at[pl.ds(base, BPW)], midx_v)

    for g in range(NG):
        pltpu.async_copy(utab_hbm.at[uidx_v.at[pl.ds(g * GCH, GCH)]],
                         urows_v.at[pl.ds(g * GCH, GCH)], sem_u)
        pltpu.async_copy(mtab_hbm.at[midx_v.at[pl.ds(g * GCH, GCH)]],
                         mrows_v.at[pl.ds(g * GCH, GCH)], sem_m)
    for g in range(NG):
        pltpu.make_async_copy(utab_hbm.at[uidx_v.at[pl.ds(g * GCH, GCH)]],
                              urows_v.at[pl.ds(g * GCH, GCH)], sem_u).wait()
        pltpu.make_async_copy(mtab_hbm.at[midx_v.at[pl.ds(g * GCH, GCH)]],
                              mrows_v.at[pl.ds(g * GCH, GCH)], sem_m).wait()

    lanes = lax.iota(jnp.int32, 16)

    def row_body(b, carry):
        s = (urows_v[b, pl.ds(0, 16)] * mrows_v[b, pl.ds(0, 16)]
             + urows_v[b, pl.ds(16, 16)] * mrows_v[b, pl.ds(16, 16)])
        plsc.store_scatter(pt_v, [lanes * BPW + b], s)
        return carry

    lax.fori_loop(0, BPW, row_body, 0)

    def red_body(gidx, carry):
        col = gidx * 16
        acc = pt_v[pl.ds(col, 16)]
        for dd in range(1, NS):
            acc = acc + pt_v[pl.ds(dd * BPW + col, 16)]
        out_v[pl.ds(col, 16)] = acc
        return carry

    lax.fori_loop(0, BPW // 16, red_body, 0)

    pltpu.sync_copy(out_v, out_hbm.at[pl.ds(base, BPW)])


def kernel(user_idx, movie_idx, user_table, movie_table):
    return _mf_kernel(user_idx, movie_idx, user_table, movie_table)
